# baseline (speedup 1.0000x reference)
import functools

import jax
import jax.numpy as jnp
from jax import lax
from jax.experimental import pallas as pl
from jax.experimental.pallas import tpu as pltpu
from jax.experimental.pallas import tpu_sc as plsc

D_MODEL = 64
SCALE = 8.0
BBLK = 128
NUM_WORKERS = 32
NBUF = 4


def kernel(x, table):
    n_seq, seq_len = x.shape
    n_bblk = n_seq // BBLK
    n_tt = seq_len // 8

    xl = x.T.reshape(n_tt, 8, n_bblk, BBLK).transpose(0, 2, 1, 3)

    mesh = plsc.VectorSubcoreMesh(core_axis_name="c", subcore_axis_name="s")

    @functools.partial(
        pl.kernel,
        mesh=mesh,
        compiler_params=pltpu.CompilerParams(
            use_tc_tiling_on_sc=False, needs_layout_passes=False
        ),
        out_type=jax.ShapeDtypeStruct(
            (seq_len, D_MODEL // 8, n_bblk, 8, BBLK), jnp.float32
        ),
        scratch_types=[
            pltpu.VMEM((n_tt, 8, BBLK), jnp.int32),
            pltpu.VMEM((NBUF, BBLK, D_MODEL), jnp.float32),
            pltpu.VMEM((2, D_MODEL // 8, 8, BBLK), jnp.float32),
            [pltpu.SemaphoreType.DMA] * NBUF,
            [pltpu.SemaphoreType.DMA] * 2,
        ],
    )
    def emb_kernel(x_hbm, table_hbm, out_hbm, idx_v, rows_v, tp_v, gsem, ssem):
        wid = lax.axis_index("s") * 2 + lax.axis_index("c")
        pltpu.sync_copy(x_hbm.at[:, wid, :, :], idx_v)

        def idx_slice(t):
            return idx_v.at[t // 8, t % 8]

        for k in range(NBUF - 1):
            pltpu.make_async_copy(
                table_hbm.at[idx_slice(k)], rows_v.at[k], gsem[k]
            ).start()

        lanes = lax.iota(jnp.int32, 16)

        def outer(it, carry):
            for k in range(NBUF):
                t = it * NBUF + k
                kt = k % 2
                buf = rows_v.at[k]
                tbuf = tp_v.at[kt]
                pltpu.make_async_copy(
                    table_hbm.at[idx_slice(0)], buf, gsem[k]
                ).wait()

                def drain_tbuf():
                    pltpu.make_async_copy(
                        tp_v.at[kt], out_hbm.at[0, :, 0], ssem[kt]
                    ).wait()

                if k >= 2:
                    drain_tbuf()
                else:
                    pl.when(it >= 1)(drain_tbuf)

                @plsc.parallel_loop(0, BBLK, unroll=4)
                def tp_tok(b):
                    bb = jnp.full((16,), b, jnp.int32)
                    for j16 in range(D_MODEL // 16):
                        jids = j16 * 16 + lanes
                        v = buf[b, pl.ds(j16 * 16, 16)]
                        plsc.store_scatter(
                            tbuf, [jids >> 3, jids & 7, bb], v * SCALE
                        )

                pltpu.make_async_copy(
                    tbuf, out_hbm.at[t, :, wid], ssem[kt]
                ).start()

                gn = t + NBUF - 1

                @pl.when(gn < seq_len)
                def _refill():
                    pltpu.make_async_copy(
                        table_hbm.at[idx_slice(gn)],
                        rows_v.at[(k + NBUF - 1) % NBUF],
                        gsem[(k + NBUF - 1) % NBUF],
                    ).start()

            return carry

        lax.fori_loop(0, seq_len // NBUF, outer, 0)

        for kt in range(2):
            pltpu.make_async_copy(
                tp_v.at[kt], out_hbm.at[0, :, 0], ssem[kt]
            ).wait()

    out5 = emb_kernel(xl, table)
    return (
        out5.transpose(2, 4, 0, 1, 3).reshape(n_seq, seq_len, D_MODEL)
    )


# speedup vs baseline: 1.2721x; 1.2721x over previous
import functools

import jax
import jax.numpy as jnp
from jax import lax
from jax.experimental import pallas as pl
from jax.experimental.pallas import tpu as pltpu
from jax.experimental.pallas import tpu_sc as plsc

D_MODEL = 64
SCALE = 8.0
BBLK = 128
NUM_WORKERS = 32
NBUF = 4
NT = 7812
TAIL0 = NT * 128


def _transpose_table(tT):
    mesh = plsc.VectorSubcoreMesh(core_axis_name="c", subcore_axis_name="s")

    @functools.partial(
        pl.kernel,
        mesh=mesh,
        compiler_params=pltpu.CompilerParams(
            use_tc_tiling_on_sc=True, needs_layout_passes=False
        ),
        out_type=jax.ShapeDtypeStruct((TAIL0 // 2 + 32, 128), jnp.float32),
        scratch_types=[
            pltpu.VMEM((3, D_MODEL, 128), jnp.float32),
            pltpu.VMEM((2, D_MODEL, 128), jnp.float32),
            [pltpu.SemaphoreType.DMA] * 3,
            [pltpu.SemaphoreType.DMA] * 2,
        ],
    )
    def tk(tT_hbm, t2_hbm, inb, outb, gsem, ssem):
        wid = lax.axis_index("s") * 2 + lax.axis_index("c")
        lanes = lax.iota(jnp.int32, 16)

        def tile_of(i):
            return i * NUM_WORKERS + wid

        def read(i, k):
            c0 = pl.multiple_of(tile_of(i) * 128, 128)
            pltpu.make_async_copy(
                tT_hbm.at[:, pl.ds(c0, 128)], inb.at[k], gsem[k]
            ).start()

        for k in range(2):
            read(k, k)

        def body(i, i3, k):
            kb = k % 3
            ob = k % 2

            @pl.when(tile_of(i) < NT)
            def _():
                pltpu.make_async_copy(
                    tT_hbm.at[:, pl.ds(0, 128)], inb.at[kb], gsem[kb]
                ).wait()

                def drain():
                    pltpu.make_async_copy(
                        outb.at[ob], t2_hbm.at[pl.ds(0, D_MODEL)], ssem[ob]
                    ).wait()

                if k >= 2:
                    drain()
                else:
                    pl.when(i3 >= 1)(drain)

                @plsc.parallel_loop(0, D_MODEL, unroll=4)
                def tp(j):
                    jb = jnp.full((16,), j, jnp.int32)
                    for m in range(8):
                        q_ids = (m * 16 + lanes) >> 1
                        c_ids = ((m * 16 + lanes) & 1) * D_MODEL + jb
                        v = inb[kb, j, pl.ds(m * 16, 16)]
                        plsc.store_scatter(outb.at[ob], [q_ids, c_ids], v)

                pltpu.make_async_copy(
                    outb.at[ob],
                    t2_hbm.at[pl.ds(pl.multiple_of(tile_of(i) * 64, 64), 64)],
                    ssem[ob],
                ).start()

                @pl.when(tile_of(i + 2) < NT)
                def _next():
                    read(i + 2, (k + 2) % 3)

        def outer(i3, carry):
            for k in range(6):
                body(i3 * 6 + k, i3, k)
            return carry

        lax.fori_loop(0, (NT // NUM_WORKERS + 6) // 6 + 1, outer, 0)

        for ob in range(2):
            pltpu.make_async_copy(
                outb.at[ob], t2_hbm.at[pl.ds(0, D_MODEL)], ssem[ob]
            ).wait()

    return tk(tT)


def _gather(xl, t2v, tail):
    n_tt = xl.shape[0]
    n_bblk = xl.shape[1]
    seq_len = n_tt * 8
    mesh = plsc.VectorSubcoreMesh(core_axis_name="c", subcore_axis_name="s")

    @functools.partial(
        pl.kernel,
        mesh=mesh,
        compiler_params=pltpu.CompilerParams(
            use_tc_tiling_on_sc=False, needs_layout_passes=False
        ),
        out_type=jax.ShapeDtypeStruct(
            (seq_len, D_MODEL // 8, n_bblk, 8, BBLK), jnp.float32
        ),
        scratch_types=[
            pltpu.VMEM((n_tt, 8, BBLK), jnp.int32),
            pltpu.VMEM((NBUF, BBLK, D_MODEL), jnp.float32),
            pltpu.VMEM((2, D_MODEL // 8, 8, BBLK + 1), jnp.float32),
            pltpu.VMEM((D_MODEL, D_MODEL), jnp.float32),
            pltpu.SMEM((NBUF,), jnp.int32),
            [pltpu.SemaphoreType.DMA] * NBUF,
            [pltpu.SemaphoreType.DMA] * 2,
        ],
    )
    def gk(
        x_hbm, t2_hbm, tail_hbm, out_hbm,
        idx_v, rows_v, tp_v, tail_v, smax, gsem, ssem,
    ):
        wid = lax.axis_index("s") * 2 + lax.axis_index("c")
        pltpu.sync_copy(x_hbm.at[:, wid, :, :], idx_v)
        pltpu.sync_copy(tail_hbm, tail_v)
        lanes = lax.iota(jnp.int32, 16)

        def idx_slice(t):
            return idx_v.at[t // 8, t % 8]

        def prep_fire(t, slot):
            row = idx_slice(t)
            acc = jnp.zeros((16,), jnp.int32)
            for v8 in range(BBLK // 16):
                acc = jnp.maximum(acc, row[pl.ds(v8 * 16, 16)])
            smax[slot] = lax.reduce_max(acc, (0,))
            pltpu.make_async_copy(
                t2_hbm.at[row], rows_v.at[slot], gsem[slot]
            ).start()

        for k in range(NBUF - 1):
            prep_fire(k, k)

        def outer(it, carry):
            for k in range(NBUF):
                t = it * NBUF + k
                kt = k % 2
                buf = rows_v.at[k]
                tbuf = tp_v.at[kt]
                pltpu.make_async_copy(
                    t2_hbm.at[idx_slice(0)], buf, gsem[k]
                ).wait()

                def drain_tbuf():
                    pltpu.make_async_copy(
                        tp_v.at[kt].at[:, :, pl.ds(0, BBLK)],
                        out_hbm.at[0, :, 0],
                        ssem[kt],
                    ).wait()

                if k >= 2:
                    drain_tbuf()
                else:
                    pl.when(it >= 1)(drain_tbuf)

                @plsc.parallel_loop(0, BBLK, unroll=4)
                def tp_tok(b):
                    bb = jnp.full((16,), b, jnp.int32)
                    for j16 in range(D_MODEL // 16):
                        jids = j16 * 16 + lanes
                        v = buf[b, pl.ds(j16 * 16, 16)]
                        plsc.store_scatter(
                            tbuf, [jids >> 3, jids & 7, bb], v * SCALE
                        )

                @pl.when(smax[k] >= TAIL0)
                def _tail_fix():
                    for b16 in range(BBLK // 16):
                        iv = idx_v[t // 8, t % 8, pl.ds(b16 * 16, 16)]
                        m = iv >= TAIL0
                        trr = iv - TAIL0
                        blane = b16 * 16 + lanes

                        @plsc.parallel_loop(0, D_MODEL, unroll=2)
                        def fix(j):
                            jb = jnp.full((16,), j, jnp.int32)
                            v = plsc.load_gather(tail_v, [trr, jb], mask=m)
                            plsc.store_scatter(
                                tbuf,
                                [jb >> 3, jb & 7, blane],
                                v * SCALE,
                                mask=m,
                            )

                pltpu.make_async_copy(
                    tbuf.at[:, :, pl.ds(0, BBLK)],
                    out_hbm.at[t, :, wid],
                    ssem[kt],
                ).start()

                gn = t + NBUF - 1

                @pl.when(gn < seq_len)
                def _refill():
                    prep_fire(gn, (k + NBUF - 1) % NBUF)

            return carry

        lax.fori_loop(0, seq_len // NBUF, outer, 0)

        for kt in range(2):
            pltpu.make_async_copy(
                tp_v.at[kt].at[:, :, pl.ds(0, BBLK)],
                out_hbm.at[0, :, 0],
                ssem[kt],
            ).wait()

    return gk(xl, t2v, tail)


def kernel(x, table):
    n_seq, seq_len = x.shape
    n_bblk = n_seq // BBLK
    n_tt = seq_len // 8

    xl = x.T.reshape(n_tt, 8, n_bblk, BBLK).transpose(0, 2, 1, 3)
    tT = jnp.swapaxes(table, 0, 1)
    tail = table[TAIL0:]

    t2 = _transpose_table(tT)
    t2v = t2.reshape(2 * (TAIL0 // 2 + 32), D_MODEL)
    out5 = _gather(xl, t2v, tail)
    return out5.transpose(2, 4, 0, 1, 3).reshape(n_seq, seq_len, D_MODEL)
